# baseline (speedup 1.0000x reference)
import functools

import jax
import jax.numpy as jnp
from jax import lax
from jax.experimental import pallas as pl
from jax.experimental.pallas import tpu as pltpu
from jax.experimental.pallas import tpu_sc as plsc

N = 10000
E = 320000
D = 128

NC = 2
NS = 16
NW = NC * NS
CH = 128
CHP = 80
E_PAD = NW * CHP * CH
R_TILE = 640
R_LAST = N - (NS - 1) * R_TILE
N_ACC = N + 8

_sc_mesh = plsc.VectorSubcoreMesh(
    core_axis_name="c", subcore_axis_name="s", num_cores=NC, num_subcores=NS)


@functools.partial(
    pl.kernel,
    out_type=jax.ShapeDtypeStruct((NC, N, D), jnp.float32),
    mesh=_sc_mesh,
    scratch_types=[
        pltpu.VMEM((CHP, CH), jnp.int32),
        pltpu.VMEM((CH,), jnp.int32),
        pltpu.VMEM((CH,), jnp.int32),
        pltpu.VMEM((CH, D), jnp.float32),
        pltpu.VMEM((CH, D), jnp.float32),
        pltpu.VMEM_SHARED((N_ACC, D), jnp.float32),
        pltpu.SemaphoreType.DMA,
        pltpu.SemaphoreType.DMA,
        pltpu.SemaphoreType.DMA,
        pltpu.SemaphoreType.DMA,
    ],
)
def _agg_kernel(srcp_hbm, dstf_hbm, hs_hbm, zeros_hbm, out_hbm,
                srcb, dstd_a, dstd_b, rows_a, rows_b, acc,
                sem_a, sem_b, sem_da, sem_db):
    c = lax.axis_index("c")
    s = lax.axis_index("s")
    wid = c * NS + s
    r0 = s * R_TILE
    ebase = wid * (CHP * CH)

    pltpu.sync_copy(srcp_hbm.at[wid], srcb)

    def _init(nr):
        @pl.when(c == 0)
        def _():
            pltpu.sync_copy(hs_hbm.at[pl.ds(r0, nr)], acc.at[pl.ds(r0, nr)])

        @pl.when(c != 0)
        def _():
            pltpu.sync_copy(zeros_hbm.at[pl.ds(0, nr)], acc.at[pl.ds(r0, nr)])

    @pl.when(s < NS - 1)
    def _():
        _init(R_TILE)

    @pl.when(s == NS - 1)
    def _():
        _init(R_LAST)

    plsc.subcore_barrier()

    def _fetch(j, rows, sem, dstd, sem_d):
        pltpu.async_copy(dstf_hbm.at[pl.ds(ebase + j * CH, CH)], dstd, sem_d)
        pltpu.async_copy(hs_hbm.at[srcb.at[j]], rows, sem)

    def _fwait(rows, sem, dstd, sem_d):
        pltpu.make_async_copy(dstf_hbm.at[pl.ds(0, CH)], dstd, sem_d).wait()
        pltpu.make_async_copy(hs_hbm.at[srcb.at[0]], rows, sem).wait()

    def _scat(rows, dstd):
        pltpu.sync_copy(rows, acc.at[dstd], add=True)

    _fetch(0, rows_a, sem_a, dstd_a, sem_da)

    def body(jj, carry):
        a = 2 * jj
        _fetch(a + 1, rows_b, sem_b, dstd_b, sem_db)
        _fwait(rows_a, sem_a, dstd_a, sem_da)
        _scat(rows_a, dstd_a)

        @pl.when(jj < CHP // 2 - 1)
        def _():
            _fetch(a + 2, rows_a, sem_a, dstd_a, sem_da)

        _fwait(rows_b, sem_b, dstd_b, sem_db)
        _scat(rows_b, dstd_b)
        return carry

    lax.fori_loop(0, CHP // 2, body, 0)
    plsc.subcore_barrier()

    @pl.when(s < NS - 1)
    def _():
        pltpu.sync_copy(acc.at[pl.ds(r0, R_TILE)],
                        out_hbm.at[c, pl.ds(r0, R_TILE)])

    @pl.when(s == NS - 1)
    def _():
        pltpu.sync_copy(acc.at[pl.ds(r0, R_LAST)],
                        out_hbm.at[c, pl.ds(r0, R_LAST)])


BR = 1000


def _disq(degp_ref):
    deg = degp_ref[0, :, 0:1] + degp_ref[1, :, 0:1]
    return lax.rsqrt(deg)


def _tc1_body(degp_ref, x_ref, w_ref, hs_ref):
    h = jnp.dot(x_ref[:], w_ref[:], preferred_element_type=jnp.float32)
    hs_ref[:] = h * _disq(degp_ref)


def _tc2_body(degp_ref, aggp_ref, b_ref, w_ref, hs_ref):
    dis = _disq(degp_ref)
    agg = aggp_ref[0] + aggp_ref[1]
    o1 = jnp.maximum(agg * dis + b_ref[:], 0.0)
    h = jnp.dot(o1, w_ref[:], preferred_element_type=jnp.float32)
    hs_ref[:] = h * dis


def _tc3_body(degp_ref, aggp_ref, b_ref, out_ref):
    agg = aggp_ref[0] + aggp_ref[1]
    out_ref[:] = agg * _disq(degp_ref) + b_ref[:]


_degp_spec = pl.BlockSpec((NC, BR, D), lambda i: (0, i, 0))
_aggp_spec = pl.BlockSpec((NC, BR, D), lambda i: (0, i, 0))
_row_spec = pl.BlockSpec((BR, D), lambda i: (i, 0))
_w_spec = pl.BlockSpec((D, D), lambda i: (0, 0))
_b_spec = pl.BlockSpec((1, D), lambda i: (0, 0))
_out_sds = jax.ShapeDtypeStruct((N, D), jnp.float32)

_tc1 = pl.pallas_call(
    _tc1_body, grid=(N // BR,),
    in_specs=[_degp_spec, _row_spec, _w_spec],
    out_specs=_row_spec, out_shape=_out_sds)

_tc2 = pl.pallas_call(
    _tc2_body, grid=(N // BR,),
    in_specs=[_degp_spec, _aggp_spec, _b_spec, _w_spec],
    out_specs=_row_spec, out_shape=_out_sds)

_tc3 = pl.pallas_call(
    _tc3_body, grid=(N // BR,),
    in_specs=[_degp_spec, _aggp_spec, _b_spec],
    out_specs=_row_spec, out_shape=_out_sds)


def kernel(x, edge_index, W1, b1, W2, b2):
    pad = E_PAD - E
    srcp = jnp.concatenate(
        [edge_index[0], jnp.zeros((pad,), jnp.int32)]).reshape(NW, CHP, CH)
    dstf = jnp.concatenate(
        [edge_index[1], jnp.full((pad,), N, jnp.int32)])
    ones_n = jnp.ones((N, D), jnp.float32)
    zeros_row = jnp.zeros((R_TILE, D), jnp.float32)

    degp = _agg_kernel(srcp, dstf, ones_n, zeros_row)
    hs1 = _tc1(degp, x, W1)
    aggp1 = _agg_kernel(srcp, dstf, hs1, zeros_row)
    hs2 = _tc2(degp, aggp1, b1.reshape(1, D), W2)
    aggp2 = _agg_kernel(srcp, dstf, hs2, zeros_row)
    return _tc3(degp, aggp2, b2.reshape(1, D))
